# baseline (speedup 1.0000x reference)
import jax
import jax.numpy as jnp
from jax.experimental import pallas as pl
from jax.experimental.pallas import tpu as pltpu

_BM = 400


def _gcn_block(adj_ref, xs_ref, h0s_ref, w_ref, out_ref):
    a = adj_ref[...].astype(jnp.bfloat16)
    xb = xs_ref[...].astype(jnp.bfloat16)
    hi = jnp.dot(a, xb, preferred_element_type=jnp.float32)
    support = hi + h0s_ref[...]
    out_ref[...] = jnp.dot(support, w_ref[...],
                           preferred_element_type=jnp.float32)


def kernel(x, adj, h0, weight, lamda, alpha, layer_idx):
    n, d_in = x.shape
    d_out = weight.shape[1]
    lamda = jnp.asarray(lamda, jnp.float32)
    alpha = jnp.asarray(alpha, jnp.float32)
    layer_f = jnp.asarray(layer_idx, jnp.float32)
    theta = jnp.log(lamda / layer_f + 1.0)
    wprime = theta * weight + (1.0 - theta) * jnp.eye(d_in, d_out,
                                                      dtype=weight.dtype)
    xs = (1.0 - alpha) * x
    h0s = alpha * h0

    grid = (n // _BM,)
    return pl.pallas_call(
        _gcn_block,
        grid=grid,
        in_specs=[
            pl.BlockSpec((_BM, n), lambda i: (i, 0)),
            pl.BlockSpec((n, d_in), lambda i: (0, 0)),
            pl.BlockSpec((_BM, d_in), lambda i: (i, 0)),
            pl.BlockSpec((d_in, d_out), lambda i: (0, 0)),
        ],
        out_specs=pl.BlockSpec((_BM, d_out), lambda i: (i, 0)),
        out_shape=jax.ShapeDtypeStruct((n, d_out), jnp.float32),
        compiler_params=pltpu.CompilerParams(
            dimension_semantics=("arbitrary",),
        ),
    )(adj, xs, h0s, wprime)
